# Initial kernel scaffold; baseline (speedup 1.0000x reference)
import functools

import jax
import jax.numpy as jnp
from jax import lax
from jax.experimental import pallas as pl
from jax.experimental.pallas import tpu as pltpu
from jax.experimental.pallas import tpu_sc as plsc

B, C, H, W = 16, 384, 24, 24
HW = H * W
R = B * C
L = 16
NC, NS = 2, 16
NW = NC * NS
TASKS = 2 * C
TPT = TASKS // NW


def _index_plan_kernel(bn1r_ref, bn1c_ref, bn2r_ref, bn2c_ref, th_ref,
                       idx1_ref, sel1_ref, idx2_ref, sel2_ref):
    f32 = jnp.float32
    th = th_ref[...]
    ii = lax.broadcasted_iota(jnp.int32, (C, C), 0)
    jj = lax.broadcasted_iota(jnp.int32, (C, C), 1)
    tri = jj <= ii
    iota_c_col = lax.broadcasted_iota(jnp.int32, (C, 1), 0)
    iota_b_row = lax.broadcasted_iota(jnp.int32, (C, L), 1)

    def plan(bn_self_r, bn_self_c, bn_other_r, bn_other_c):
        keep_c = jnp.abs(bn_self_c) >= th
        nk_r = jnp.where(jnp.abs(bn_self_r) >= th, 0.0, 1.0)
        rank = jnp.sum(jnp.where(tri, jnp.broadcast_to(nk_r, (C, C)), 0.0),
                       axis=1, keepdims=True) - 1.0
        rank = jnp.clip(rank, 0.0, float(C - 1))
        ao_r = jnp.abs(bn_other_r)
        ao_c = jnp.abs(bn_other_c)
        bigger = (ao_c > ao_r) | ((ao_c == ao_r) & (ii < jj))
        pos_r = jnp.sum(jnp.where(bigger, 1.0, 0.0), axis=0,
                        keepdims=True)
        onehot = pos_r == rank
        src = jnp.sum(jnp.where(onehot, jj.astype(f32), 0.0),
                      axis=1, keepdims=True)
        chan = jnp.where(keep_c, iota_c_col, src.astype(jnp.int32))
        idx = jnp.broadcast_to(chan, (C, L)) + iota_b_row * C
        sel = jnp.broadcast_to(jnp.where(keep_c, 0, 1), (C, L))
        return idx, sel

    idx1, sel1 = plan(bn1r_ref[...], bn1c_ref[...], bn2r_ref[...], bn2c_ref[...])
    idx2, sel2 = plan(bn2r_ref[...], bn2c_ref[...], bn1r_ref[...], bn1c_ref[...])
    idx1_ref[...] = idx1
    sel1_ref[...] = sel1
    idx2_ref[...] = idx2
    sel2_ref[...] = sel2


def _index_plan(bn1, bn2, th):
    bn1r = bn1.reshape(1, C)
    bn1c = bn1.reshape(C, 1)
    bn2r = bn2.reshape(1, C)
    bn2c = bn2.reshape(C, 1)
    th_arr = jnp.asarray(th, jnp.float32).reshape(1, 1)
    out_shape = (
        jax.ShapeDtypeStruct((C, L), jnp.int32),
        jax.ShapeDtypeStruct((C, L), jnp.int32),
        jax.ShapeDtypeStruct((C, L), jnp.int32),
        jax.ShapeDtypeStruct((C, L), jnp.int32),
    )
    return pl.pallas_call(_index_plan_kernel, out_shape=out_shape)(
        bn1r, bn1c, bn2r, bn2c, th_arr)


def _sc_exchange_body(f0_hbm, f1_hbm, idx1_hbm, sel1_hbm, idx2_hbm, sel2_hbm,
                      out1_hbm, out2_hbm, idx_v, sel_v, buf0, buf1,
                      gsem, ssem0, ssem1):
    wid = lax.axis_index("s") * NC + lax.axis_index("c")
    iota = lax.iota(jnp.int32, L)

    def half(self_hbm, other_hbm, idx_hbm, sel_hbm, out_hbm, cbase):
        pltpu.sync_copy(idx_hbm.at[pl.ds(cbase, TPT)], idx_v)
        pltpu.sync_copy(sel_hbm.at[pl.ds(cbase, TPT)], sel_v)
        for k in range(TPT):
            c = cbase + k
            buf = buf0 if k % 2 == 0 else buf1
            ssem = ssem0 if k % 2 == 0 else ssem1
            if k >= 2:
                pltpu.make_async_copy(buf, out_hbm.at[iota], ssem).wait()
            sflag = jnp.max(sel_v[k])
            irow = idx_v[k]
            dst = iota * C + c

            @pl.when(sflag > 0)
            def _():
                pltpu.async_copy(other_hbm.at[irow], buf, gsem).wait()

            @pl.when(sflag == 0)
            def _():
                pltpu.async_copy(self_hbm.at[irow], buf, gsem).wait()

            pltpu.async_copy(buf, out_hbm.at[dst], ssem)
        pltpu.make_async_copy(buf0, out_hbm.at[iota], ssem0).wait()
        pltpu.make_async_copy(buf1, out_hbm.at[iota], ssem1).wait()

    @pl.when(wid < NS)
    def _():
        half(f0_hbm, f1_hbm, idx1_hbm, sel1_hbm, out1_hbm, wid * TPT)

    @pl.when(wid >= NS)
    def _():
        half(f1_hbm, f0_hbm, idx2_hbm, sel2_hbm, out2_hbm, (wid - NS) * TPT)


@functools.partial(
    pl.kernel,
    out_type=(jax.ShapeDtypeStruct((R, HW), jnp.float32),
              jax.ShapeDtypeStruct((R, HW), jnp.float32)),
    mesh=plsc.VectorSubcoreMesh(core_axis_name="c", subcore_axis_name="s"),
    scratch_types=[
        pltpu.VMEM((TPT, L), jnp.int32),
        pltpu.VMEM((TPT, L), jnp.int32),
        pltpu.VMEM((L, HW), jnp.float32),
        pltpu.VMEM((L, HW), jnp.float32),
        pltpu.SemaphoreType.DMA,
        pltpu.SemaphoreType.DMA,
        pltpu.SemaphoreType.DMA,
    ],
)
def _sc_exchange(f0_hbm, f1_hbm, idx1_hbm, sel1_hbm, idx2_hbm, sel2_hbm,
                 out1_hbm, out2_hbm, idx_v, sel_v, buf0, buf1,
                 gsem, ssem0, ssem1):
    _sc_exchange_body(f0_hbm, f1_hbm, idx1_hbm, sel1_hbm, idx2_hbm, sel2_hbm,
                      out1_hbm, out2_hbm, idx_v, sel_v, buf0, buf1,
                      gsem, ssem0, ssem1)


def kernel(features_0, features_1, bn1_weight, bn2_weight, bn_threshold):
    idx1, sel1, idx2, sel2 = _index_plan(bn1_weight, bn2_weight, bn_threshold)
    f0 = features_0.reshape(R, HW)
    f1 = features_1.reshape(R, HW)
    out1, out2 = _sc_exchange(f0, f1, idx1, sel1, idx2, sel2)
    return (out1.reshape(B, C, H, W), out2.reshape(B, C, H, W))


# speedup vs baseline: 1.3873x; 1.3873x over previous
import functools

import jax
import jax.numpy as jnp
from jax import lax
from jax.experimental import pallas as pl
from jax.experimental.pallas import tpu as pltpu
from jax.experimental.pallas import tpu_sc as plsc

B, C, H, W = 16, 384, 24, 24
HW = H * W
R = B * C
L = 16
NC, NS = 2, 16
NW = NC * NS
TASKS = 2 * C
TPT = TASKS // NW


def _index_plan_kernel(bn1r_ref, bn1c_ref, bn2r_ref, bn2c_ref, th_ref,
                       idx1_ref, sel1_ref, idx2_ref, sel2_ref):
    f32 = jnp.float32
    th = th_ref[...]
    ii = lax.broadcasted_iota(jnp.int32, (C, C), 0)
    jj = lax.broadcasted_iota(jnp.int32, (C, C), 1)
    tri = jj <= ii
    iota_c_col = lax.broadcasted_iota(jnp.int32, (C, 1), 0)
    iota_b_row = lax.broadcasted_iota(jnp.int32, (C, L), 1)

    def plan(bn_self_r, bn_self_c, bn_other_r, bn_other_c):
        keep_c = jnp.abs(bn_self_c) >= th
        nk_r = jnp.where(jnp.abs(bn_self_r) >= th, 0.0, 1.0)
        rank = jnp.sum(jnp.where(tri, jnp.broadcast_to(nk_r, (C, C)), 0.0),
                       axis=1, keepdims=True) - 1.0
        rank = jnp.clip(rank, 0.0, float(C - 1))
        ao_r = jnp.abs(bn_other_r)
        ao_c = jnp.abs(bn_other_c)
        bigger = (ao_c > ao_r) | ((ao_c == ao_r) & (ii < jj))
        pos_r = jnp.sum(jnp.where(bigger, 1.0, 0.0), axis=0,
                        keepdims=True)
        onehot = pos_r == rank
        src = jnp.sum(jnp.where(onehot, jj.astype(f32), 0.0),
                      axis=1, keepdims=True)
        chan = jnp.where(keep_c, iota_c_col, src.astype(jnp.int32))
        idx = jnp.broadcast_to(chan, (C, L)) + iota_b_row * C
        sel = jnp.broadcast_to(jnp.where(keep_c, 0, 1), (C, L))
        return idx, sel

    idx1, sel1 = plan(bn1r_ref[...], bn1c_ref[...], bn2r_ref[...], bn2c_ref[...])
    idx2, sel2 = plan(bn2r_ref[...], bn2c_ref[...], bn1r_ref[...], bn1c_ref[...])
    idx1_ref[...] = idx1
    sel1_ref[...] = sel1
    idx2_ref[...] = idx2
    sel2_ref[...] = sel2


def _index_plan(bn1, bn2, th):
    bn1r = bn1.reshape(1, C)
    bn1c = bn1.reshape(C, 1)
    bn2r = bn2.reshape(1, C)
    bn2c = bn2.reshape(C, 1)
    th_arr = jnp.asarray(th, jnp.float32).reshape(1, 1)
    out_shape = (
        jax.ShapeDtypeStruct((C, L), jnp.int32),
        jax.ShapeDtypeStruct((C, L), jnp.int32),
        jax.ShapeDtypeStruct((C, L), jnp.int32),
        jax.ShapeDtypeStruct((C, L), jnp.int32),
    )
    return pl.pallas_call(_index_plan_kernel, out_shape=out_shape)(
        bn1r, bn1c, bn2r, bn2c, th_arr)


def _sc_exchange_body(f0_hbm, f1_hbm, idx1_hbm, sel1_hbm, idx2_hbm, sel2_hbm,
                      out1_hbm, out2_hbm, idx_v, sel_v, buf0, buf1,
                      gsem, ssem0, ssem1):
    wid = lax.axis_index("s") * NC + lax.axis_index("c")
    iota = lax.iota(jnp.int32, L)

    def half(self_hbm, other_hbm, idx_hbm, sel_hbm, out_hbm, cbase):
        pltpu.sync_copy(idx_hbm.at[pl.ds(cbase, TPT)], idx_v)
        pltpu.sync_copy(sel_hbm.at[pl.ds(cbase, TPT)], sel_v)
        for k in range(TPT):
            c = cbase + k
            buf = buf0 if k % 2 == 0 else buf1
            ssem = ssem0 if k % 2 == 0 else ssem1
            if k >= 2:
                pltpu.make_async_copy(buf, out_hbm.at[iota], ssem).wait()
            sflag = sel_v[k][0]
            irow = idx_v[k]
            dst = iota * C + c

            @pl.when(sflag > 0)
            def _():
                pltpu.async_copy(other_hbm.at[irow], buf, gsem).wait()

            @pl.when(sflag == 0)
            def _():
                pltpu.async_copy(self_hbm.at[irow], buf, gsem).wait()

            pltpu.async_copy(buf, out_hbm.at[dst], ssem)
        pltpu.make_async_copy(buf0, out_hbm.at[iota], ssem0).wait()
        pltpu.make_async_copy(buf1, out_hbm.at[iota], ssem1).wait()

    @pl.when(wid < NS)
    def _():
        half(f0_hbm, f1_hbm, idx1_hbm, sel1_hbm, out1_hbm, wid * TPT)

    @pl.when(wid >= NS)
    def _():
        half(f1_hbm, f0_hbm, idx2_hbm, sel2_hbm, out2_hbm, (wid - NS) * TPT)


@functools.lru_cache(maxsize=1)
def _sc_exchange():
    return pl.kernel(
        _sc_exchange_body,
        out_type=(jax.ShapeDtypeStruct((R, HW), jnp.float32),
                  jax.ShapeDtypeStruct((R, HW), jnp.float32)),
        mesh=plsc.VectorSubcoreMesh(core_axis_name="c", subcore_axis_name="s"),
        scratch_types=[
            pltpu.VMEM((TPT, L), jnp.int32),
            pltpu.VMEM((TPT, L), jnp.int32),
            pltpu.VMEM((L, HW), jnp.float32),
            pltpu.VMEM((L, HW), jnp.float32),
            pltpu.SemaphoreType.DMA,
            pltpu.SemaphoreType.DMA,
            pltpu.SemaphoreType.DMA,
        ],
        compiler_params=pltpu.CompilerParams(use_tc_tiling_on_sc=False),
    )


def kernel(features_0, features_1, bn1_weight, bn2_weight, bn_threshold):
    idx1, sel1, idx2, sel2 = _index_plan(bn1_weight, bn2_weight, bn_threshold)
    f0 = features_0.reshape(R, HW)
    f1 = features_1.reshape(R, HW)
    out1, out2 = _sc_exchange()(f0, f1, idx1, sel1, idx2, sel2)
    return (out1.reshape(B, C, H, W), out2.reshape(B, C, H, W))


# speedup vs baseline: 1.8402x; 1.3265x over previous
import functools

import jax
import jax.numpy as jnp
from jax import lax
from jax.experimental import pallas as pl
from jax.experimental.pallas import tpu as pltpu
from jax.experimental.pallas import tpu_sc as plsc

B, C, H, W = 16, 384, 24, 24
P = B * H * W
L = 16
NC, NS = 2, 16
NW = NC * NS
RPT = P // NW
BLK = 16
NBLK = RPT // BLK
NCH = C // L


def _index_plan_kernel(bn1r_ref, bn1c_ref, bn2r_ref, bn2c_ref, th_ref,
                       rb1_ref, c1_ref, rb2_ref, c2_ref):
    f32 = jnp.float32
    th = th_ref[...]
    ia0 = lax.broadcasted_iota(jnp.int32, (C, C), 0)
    ia1 = lax.broadcasted_iota(jnp.int32, (C, C), 1)
    iota_row = lax.broadcasted_iota(jnp.int32, (1, C), 1)

    def plan(bn_self_r, bn_self_c, bn_other_r, bn_other_c, self_base, other_base):
        keep_r = jnp.abs(bn_self_r) >= th
        nk_c = jnp.where(jnp.abs(bn_self_c) >= th, 0.0, 1.0)
        rank = jnp.sum(jnp.where(ia0 <= ia1, jnp.broadcast_to(nk_c, (C, C)), 0.0),
                       axis=0, keepdims=True) - 1.0
        rank = jnp.clip(rank, 0.0, float(C - 1))
        ao_r = jnp.abs(bn_other_r)
        ao_c = jnp.abs(bn_other_c)
        bigger = (ao_r > ao_c) | ((ao_r == ao_c) & (ia1 < ia0))
        pos = jnp.sum(jnp.where(bigger, 1.0, 0.0), axis=1,
                      keepdims=True)
        onehot = pos == rank
        src = jnp.sum(jnp.where(onehot, ia0.astype(f32), 0.0),
                      axis=0, keepdims=True)
        col = jnp.where(keep_r, iota_row, src.astype(jnp.int32))
        rbase = jnp.where(keep_r, self_base, other_base)
        return rbase, col

    rb1, c1 = plan(bn1r_ref[...], bn1c_ref[...], bn2r_ref[...], bn2c_ref[...],
                   0, BLK)
    rb2, c2 = plan(bn2r_ref[...], bn2c_ref[...], bn1r_ref[...], bn1c_ref[...],
                   BLK, 0)
    rb1_ref[...] = rb1
    c1_ref[...] = c1
    rb2_ref[...] = rb2
    c2_ref[...] = c2


def _index_plan(bn1, bn2, th):
    th_arr = jnp.asarray(th, jnp.float32).reshape(1, 1)
    out_shape = (jax.ShapeDtypeStruct((1, C), jnp.int32),) * 4
    return pl.pallas_call(_index_plan_kernel, out_shape=out_shape)(
        bn1.reshape(1, C), bn1.reshape(C, 1),
        bn2.reshape(1, C), bn2.reshape(C, 1), th_arr)


def _sc_exchange_body(f0_hbm, f1_hbm, rb1_hbm, c1_hbm, rb2_hbm, c2_hbm,
                      out1_hbm, out2_hbm, rb1_v, c1_v, rb2_v, c2_v,
                      cat, ob1, ob2):
    wid = lax.axis_index("s") * NC + lax.axis_index("c")
    base = wid * RPT
    pltpu.sync_copy(rb1_hbm, rb1_v)
    pltpu.sync_copy(c1_hbm, c1_v)
    pltpu.sync_copy(rb2_hbm, rb2_v)
    pltpu.sync_copy(c2_hbm, c2_v)

    def block(blk, carry):
        r0 = base + blk * BLK
        pltpu.sync_copy(f0_hbm.at[pl.ds(r0, BLK)], cat.at[pl.ds(0, BLK)])
        pltpu.sync_copy(f1_hbm.at[pl.ds(r0, BLK)], cat.at[pl.ds(BLK, BLK)])
        for j in range(NCH):
            rbj1 = rb1_v[j]
            cj1 = c1_v[j]
            rbj2 = rb2_v[j]
            cj2 = c2_v[j]
            for r in range(BLK):
                ob1[r, L * j:L * (j + 1)] = plsc.load_gather(cat, [rbj1 + r, cj1])
                ob2[r, L * j:L * (j + 1)] = plsc.load_gather(cat, [rbj2 + r, cj2])
        pltpu.sync_copy(ob1, out1_hbm.at[pl.ds(r0, BLK)])
        pltpu.sync_copy(ob2, out2_hbm.at[pl.ds(r0, BLK)])
        return carry

    lax.fori_loop(0, NBLK, block, 0)


@functools.lru_cache(maxsize=1)
def _sc_exchange():
    return pl.kernel(
        _sc_exchange_body,
        out_type=(jax.ShapeDtypeStruct((P, C), jnp.float32),
                  jax.ShapeDtypeStruct((P, C), jnp.float32)),
        mesh=plsc.VectorSubcoreMesh(core_axis_name="c", subcore_axis_name="s"),
        scratch_types=[
            pltpu.VMEM((NCH, L), jnp.int32),
            pltpu.VMEM((NCH, L), jnp.int32),
            pltpu.VMEM((NCH, L), jnp.int32),
            pltpu.VMEM((NCH, L), jnp.int32),
            pltpu.VMEM((2 * BLK, C), jnp.float32),
            pltpu.VMEM((BLK, C), jnp.float32),
            pltpu.VMEM((BLK, C), jnp.float32),
        ],
        compiler_params=pltpu.CompilerParams(needs_layout_passes=False),
    )


def kernel(features_0, features_1, bn1_weight, bn2_weight, bn_threshold):
    rb1, c1, rb2, c2 = _index_plan(bn1_weight, bn2_weight, bn_threshold)
    f0 = features_0.transpose(0, 2, 3, 1).reshape(P, C)
    f1 = features_1.transpose(0, 2, 3, 1).reshape(P, C)
    o1, o2 = _sc_exchange()(f0, f1,
                            rb1.reshape(NCH, L), c1.reshape(NCH, L),
                            rb2.reshape(NCH, L), c2.reshape(NCH, L))
    out1 = o1.reshape(B, H, W, C).transpose(0, 3, 1, 2)
    out2 = o2.reshape(B, H, W, C).transpose(0, 3, 1, 2)
    return (out1, out2)


# speedup vs baseline: 4.5379x; 2.4659x over previous
import functools

import jax
import jax.numpy as jnp
from jax import lax
from jax.experimental import pallas as pl
from jax.experimental.pallas import tpu as pltpu
from jax.experimental.pallas import tpu_sc as plsc

B, C, H, W = 16, 384, 24, 24
P = B * H * W
L = 16
NC, NS = 2, 16
NW = NC * NS
RPT = P // NW
BLK = 16
NBLK = RPT // BLK
NCH = C // L


def _index_plan_kernel(bn1r_ref, bn1c_ref, bn2r_ref, bn2c_ref, th_ref,
                       g1_ref, g2_ref):
    f32 = jnp.float32
    th = th_ref[...]
    ia0 = lax.broadcasted_iota(jnp.int32, (C, C), 0)
    ia1 = lax.broadcasted_iota(jnp.int32, (C, C), 1)
    iota_row = lax.broadcasted_iota(jnp.int32, (1, C), 1)

    def plan(bn_self_r, bn_self_c, bn_other_r, bn_other_c, self_off, other_off):
        keep_r = jnp.abs(bn_self_r) >= th
        nk_c = jnp.where(jnp.abs(bn_self_c) >= th, 0.0, 1.0)
        rank = jnp.sum(jnp.where(ia0 <= ia1, jnp.broadcast_to(nk_c, (C, C)), 0.0),
                       axis=0, keepdims=True) - 1.0
        rank = jnp.clip(rank, 0.0, float(C - 1))
        ao_r = jnp.abs(bn_other_r)
        ao_c = jnp.abs(bn_other_c)
        bigger = (ao_r > ao_c) | ((ao_r == ao_c) & (ia1 < ia0))
        pos = jnp.sum(jnp.where(bigger, 1.0, 0.0), axis=1,
                      keepdims=True)
        onehot = pos == rank
        src = jnp.sum(jnp.where(onehot, ia0.astype(f32), 0.0),
                      axis=0, keepdims=True)
        return jnp.where(keep_r, iota_row + self_off,
                         src.astype(jnp.int32) + other_off)

    g1 = plan(bn1r_ref[...], bn1c_ref[...], bn2r_ref[...], bn2c_ref[...], 0, C)
    g2 = plan(bn2r_ref[...], bn2c_ref[...], bn1r_ref[...], bn1c_ref[...], C, 0)
    g1_ref[...] = g1
    g2_ref[...] = g2


def _index_plan(bn1, bn2, th):
    th_arr = jnp.asarray(th, jnp.float32).reshape(1, 1)
    out_shape = (jax.ShapeDtypeStruct((1, C), jnp.int32),) * 2
    return pl.pallas_call(_index_plan_kernel, out_shape=out_shape)(
        bn1.reshape(1, C), bn1.reshape(C, 1),
        bn2.reshape(1, C), bn2.reshape(C, 1), th_arr)


def _sc_exchange_body(f0_hbm, f1_hbm, g1_hbm, g2_hbm,
                      out1_hbm, out2_hbm, g1_v, g2_v,
                      cat0, cat1, ob1a, ob2a, ob1b, ob2b,
                      insem0, insem1, osema, osemb):
    wid = lax.axis_index("s") * NC + lax.axis_index("c")
    base = wid * RPT
    pltpu.sync_copy(g1_hbm, g1_v)
    pltpu.sync_copy(g2_hbm, g2_v)
    iota = lax.iota(jnp.int32, L)

    def start_in(blk, cat, insem):
        r0 = base + blk * BLK
        pltpu.async_copy(f0_hbm.at[pl.ds(r0, BLK)], cat.at[:, pl.ds(0, C)], insem)
        pltpu.async_copy(f1_hbm.at[pl.ds(r0, BLK)], cat.at[:, pl.ds(C, C)], insem)

    def wait_in(cat, insem):
        pltpu.make_async_copy(f0_hbm.at[pl.ds(0, BLK)], cat.at[:, pl.ds(0, C)], insem).wait()
        pltpu.make_async_copy(f1_hbm.at[pl.ds(0, BLK)], cat.at[:, pl.ds(C, C)], insem).wait()

    def start_out(blk, o1, o2, osem):
        r0 = base + blk * BLK
        pltpu.async_copy(o1, out1_hbm.at[pl.ds(r0, BLK)], osem)
        pltpu.async_copy(o2, out2_hbm.at[pl.ds(r0, BLK)], osem)

    def wait_out(o1, o2, osem):
        pltpu.make_async_copy(o1, out1_hbm.at[pl.ds(0, BLK)], osem).wait()
        pltpu.make_async_copy(o2, out2_hbm.at[pl.ds(0, BLK)], osem).wait()

    def compute(cat, o1, o2):
        def jbody(j, carry):
            gj1 = g1_v[j]
            gj2 = g2_v[j]
            dcol = iota + j * L
            for r in range(BLK):
                rfull = jnp.full((L,), r, jnp.int32)
                plsc.store_scatter(o1, [rfull, dcol],
                                   plsc.load_gather(cat, [rfull, gj1]))
                plsc.store_scatter(o2, [rfull, dcol],
                                   plsc.load_gather(cat, [rfull, gj2]))
            return carry

        lax.fori_loop(0, NCH, jbody, 0)

    start_in(0, cat0, insem0)
    start_in(1, cat1, insem1)

    def superblock(sb, carry):
        blk = 2 * sb

        wait_in(cat0, insem0)

        @pl.when(sb > 0)
        def _():
            wait_out(ob1a, ob2a, osema)

        compute(cat0, ob1a, ob2a)

        @pl.when(sb < NBLK // 2 - 1)
        def _():
            start_in(blk + 2, cat0, insem0)

        start_out(blk, ob1a, ob2a, osema)

        wait_in(cat1, insem1)

        @pl.when(sb > 0)
        def _():
            wait_out(ob1b, ob2b, osemb)

        compute(cat1, ob1b, ob2b)

        @pl.when(sb < NBLK // 2 - 1)
        def _():
            start_in(blk + 3, cat1, insem1)

        start_out(blk + 1, ob1b, ob2b, osemb)
        return carry

    lax.fori_loop(0, NBLK // 2, superblock, 0)
    wait_out(ob1a, ob2a, osema)
    wait_out(ob1b, ob2b, osemb)


@functools.lru_cache(maxsize=1)
def _sc_exchange():
    return pl.kernel(
        _sc_exchange_body,
        out_type=(jax.ShapeDtypeStruct((P, C), jnp.float32),
                  jax.ShapeDtypeStruct((P, C), jnp.float32)),
        mesh=plsc.VectorSubcoreMesh(core_axis_name="c", subcore_axis_name="s"),
        scratch_types=[
            pltpu.VMEM((NCH, L), jnp.int32),
            pltpu.VMEM((NCH, L), jnp.int32),
            pltpu.VMEM((BLK, 2 * C), jnp.float32),
            pltpu.VMEM((BLK, 2 * C), jnp.float32),
            pltpu.VMEM((BLK, C), jnp.float32),
            pltpu.VMEM((BLK, C), jnp.float32),
            pltpu.VMEM((BLK, C), jnp.float32),
            pltpu.VMEM((BLK, C), jnp.float32),
            pltpu.SemaphoreType.DMA,
            pltpu.SemaphoreType.DMA,
            pltpu.SemaphoreType.DMA,
            pltpu.SemaphoreType.DMA,
        ],
        compiler_params=pltpu.CompilerParams(needs_layout_passes=False),
    )


def kernel(features_0, features_1, bn1_weight, bn2_weight, bn_threshold):
    g1, g2 = _index_plan(bn1_weight, bn2_weight, bn_threshold)
    f0 = features_0.transpose(0, 2, 3, 1).reshape(P, C)
    f1 = features_1.transpose(0, 2, 3, 1).reshape(P, C)
    o1, o2 = _sc_exchange()(f0, f1, g1.reshape(NCH, L), g2.reshape(NCH, L))
    out1 = o1.reshape(B, H, W, C).transpose(0, 3, 1, 2)
    out2 = o2.reshape(B, H, W, C).transpose(0, 3, 1, 2)
    return (out1, out2)


# speedup vs baseline: 5.1521x; 1.1354x over previous
import functools

import jax
import jax.numpy as jnp
from jax import lax
from jax.experimental import pallas as pl
from jax.experimental.pallas import tpu as pltpu
from jax.experimental.pallas import tpu_sc as plsc

B, C, H, W = 16, 384, 24, 24
P = B * H * W
L = 16
NC, NS = 2, 16
NW = NC * NS
RPT = P // NW
BLK = 16
NBLK = RPT // BLK
NCH = C // L


def _index_plan_kernel(bn1r_ref, bn1c_ref, bn2r_ref, bn2c_ref, th_ref,
                       g1_ref, g2_ref):
    f32 = jnp.float32
    th = th_ref[...]
    ia0 = lax.broadcasted_iota(jnp.int32, (C, C), 0)
    ia1 = lax.broadcasted_iota(jnp.int32, (C, C), 1)
    iota_row = lax.broadcasted_iota(jnp.int32, (1, C), 1)

    def plan(bn_self_r, bn_self_c, bn_other_r, bn_other_c, self_off, other_off):
        keep_r = jnp.abs(bn_self_r) >= th
        nk_c = jnp.where(jnp.abs(bn_self_c) >= th, 0.0, 1.0)
        rank = jnp.sum(jnp.where(ia0 <= ia1, jnp.broadcast_to(nk_c, (C, C)), 0.0),
                       axis=0, keepdims=True) - 1.0
        rank = jnp.clip(rank, 0.0, float(C - 1))
        ao_r = jnp.abs(bn_other_r)
        ao_c = jnp.abs(bn_other_c)
        bigger = (ao_r > ao_c) | ((ao_r == ao_c) & (ia1 < ia0))
        pos = jnp.sum(jnp.where(bigger, 1.0, 0.0), axis=1,
                      keepdims=True)
        onehot = pos == rank
        src = jnp.sum(jnp.where(onehot, ia0.astype(f32), 0.0),
                      axis=0, keepdims=True)
        return jnp.where(keep_r, iota_row + self_off,
                         src.astype(jnp.int32) + other_off)

    g1 = plan(bn1r_ref[...], bn1c_ref[...], bn2r_ref[...], bn2c_ref[...], 0, C)
    g2 = plan(bn2r_ref[...], bn2c_ref[...], bn1r_ref[...], bn1c_ref[...], C, 0)
    g1_ref[...] = g1
    g2_ref[...] = g2


def _index_plan(bn1, bn2, th):
    th_arr = jnp.asarray(th, jnp.float32).reshape(1, 1)
    out_shape = (jax.ShapeDtypeStruct((1, C), jnp.int32),) * 2
    return pl.pallas_call(_index_plan_kernel, out_shape=out_shape)(
        bn1.reshape(1, C), bn1.reshape(C, 1),
        bn2.reshape(1, C), bn2.reshape(C, 1), th_arr)


def _sc_exchange_body(f0_hbm, f1_hbm, g1_hbm, g2_hbm,
                      out1_hbm, out2_hbm, g1_v, g2_v,
                      cat0, cat1, ob1a, ob2a, ob1b, ob2b,
                      insem0, insem1, osema, osemb):
    wid = lax.axis_index("s") * NC + lax.axis_index("c")
    base = wid * RPT
    pltpu.sync_copy(g1_hbm, g1_v)
    pltpu.sync_copy(g2_hbm, g2_v)
    iota = lax.iota(jnp.int32, L)

    def start_in(blk, cat, insem):
        r0 = base + blk * BLK
        pltpu.async_copy(f0_hbm.at[pl.ds(r0, BLK)], cat.at[:, pl.ds(0, C)], insem)
        pltpu.async_copy(f1_hbm.at[pl.ds(r0, BLK)], cat.at[:, pl.ds(C, C)], insem)

    def wait_in(cat, insem):
        pltpu.make_async_copy(f0_hbm.at[pl.ds(0, BLK)], cat.at[:, pl.ds(0, C)], insem).wait()
        pltpu.make_async_copy(f1_hbm.at[pl.ds(0, BLK)], cat.at[:, pl.ds(C, C)], insem).wait()

    def start_out(blk, o1, o2, osem):
        r0 = base + blk * BLK
        pltpu.async_copy(o1, out1_hbm.at[pl.ds(r0, BLK)], osem)
        pltpu.async_copy(o2, out2_hbm.at[pl.ds(r0, BLK)], osem)

    def wait_out(o1, o2, osem):
        pltpu.make_async_copy(o1, out1_hbm.at[pl.ds(0, BLK)], osem).wait()
        pltpu.make_async_copy(o2, out2_hbm.at[pl.ds(0, BLK)], osem).wait()

    rfulls = [jnp.full((L,), r, jnp.int32) for r in range(BLK)]

    def compute(cat, o1, o2):
        for j in range(NCH):
            gj1 = g1_v[j]
            gj2 = g2_v[j]
            gs1 = [plsc.load_gather(cat, [rfulls[r], gj1]) for r in range(BLK)]
            for r in range(BLK):
                o1[r, L * j:L * (j + 1)] = gs1[r]
            gs2 = [plsc.load_gather(cat, [rfulls[r], gj2]) for r in range(BLK)]
            for r in range(BLK):
                o2[r, L * j:L * (j + 1)] = gs2[r]

    start_in(0, cat0, insem0)
    start_in(1, cat1, insem1)

    def superblock(sb, carry):
        blk = 2 * sb

        wait_in(cat0, insem0)

        @pl.when(sb > 0)
        def _():
            wait_out(ob1a, ob2a, osema)

        compute(cat0, ob1a, ob2a)

        @pl.when(sb < NBLK // 2 - 1)
        def _():
            start_in(blk + 2, cat0, insem0)

        start_out(blk, ob1a, ob2a, osema)

        wait_in(cat1, insem1)

        @pl.when(sb > 0)
        def _():
            wait_out(ob1b, ob2b, osemb)

        compute(cat1, ob1b, ob2b)

        @pl.when(sb < NBLK // 2 - 1)
        def _():
            start_in(blk + 3, cat1, insem1)

        start_out(blk + 1, ob1b, ob2b, osemb)
        return carry

    lax.fori_loop(0, NBLK // 2, superblock, 0)
    wait_out(ob1a, ob2a, osema)
    wait_out(ob1b, ob2b, osemb)


@functools.lru_cache(maxsize=1)
def _sc_exchange():
    return pl.kernel(
        _sc_exchange_body,
        out_type=(jax.ShapeDtypeStruct((P, C), jnp.float32),
                  jax.ShapeDtypeStruct((P, C), jnp.float32)),
        mesh=plsc.VectorSubcoreMesh(core_axis_name="c", subcore_axis_name="s"),
        scratch_types=[
            pltpu.VMEM((NCH, L), jnp.int32),
            pltpu.VMEM((NCH, L), jnp.int32),
            pltpu.VMEM((BLK, 2 * C), jnp.float32),
            pltpu.VMEM((BLK, 2 * C), jnp.float32),
            pltpu.VMEM((BLK, C), jnp.float32),
            pltpu.VMEM((BLK, C), jnp.float32),
            pltpu.VMEM((BLK, C), jnp.float32),
            pltpu.VMEM((BLK, C), jnp.float32),
            pltpu.SemaphoreType.DMA,
            pltpu.SemaphoreType.DMA,
            pltpu.SemaphoreType.DMA,
            pltpu.SemaphoreType.DMA,
        ],
        compiler_params=pltpu.CompilerParams(needs_layout_passes=False),
    )


def kernel(features_0, features_1, bn1_weight, bn2_weight, bn_threshold):
    g1, g2 = _index_plan(bn1_weight, bn2_weight, bn_threshold)
    f0 = features_0.transpose(0, 2, 3, 1).reshape(P, C)
    f1 = features_1.transpose(0, 2, 3, 1).reshape(P, C)
    o1, o2 = _sc_exchange()(f0, f1, g1.reshape(NCH, L), g2.reshape(NCH, L))
    out1 = o1.reshape(B, H, W, C).transpose(0, 3, 1, 2)
    out2 = o2.reshape(B, H, W, C).transpose(0, 3, 1, 2)
    return (out1, out2)
